# Initial kernel scaffold; baseline (speedup 1.0000x reference)
import jax
import jax.numpy as jnp
from jax.experimental import pallas as pl


def kernel(edge_index, edge_weight, features, theta_1, theta_2):
    raise NotImplementedError("write your pallas kernel here")


# speedup vs baseline: 1.0003x; 1.0003x over previous
import jax
import jax.numpy as jnp
from jax.experimental import pallas as pl
from jax.experimental.pallas import tpu as pltpu


def _combine_body(th_ref, ew_ref, v_ref, o_ref):
    t1 = th_ref[0]
    t2 = th_ref[1]
    o_ref[...] = jnp.maximum(t1 * ew_ref[...] + t2 * v_ref[...], 0.0)


def _combine(theta_1, theta_2, ew, vals):
    E = ew.shape[0]
    R = E // 128
    th = jnp.concatenate([theta_1, theta_2])
    out = pl.pallas_call(
        _combine_body,
        in_specs=[
            pl.BlockSpec(memory_space=pltpu.SMEM),
            pl.BlockSpec((R, 128), lambda: (0, 0)),
            pl.BlockSpec((R, 128), lambda: (0, 0)),
        ],
        out_specs=pl.BlockSpec((R, 128), lambda: (0, 0)),
        out_shape=jax.ShapeDtypeStruct((R, 128), jnp.float32),
    )(th, ew.reshape(R, 128), vals.reshape(R, 128))
    return out.reshape(E)


def kernel(edge_index, edge_weight, features, theta_1, theta_2):
    N = features.shape[0]
    row, col = edge_index[0], edge_index[1]
    ew_const = jax.lax.stop_gradient(edge_weight)
    X = jax.lax.stop_gradient(features)
    A = jnp.zeros((N, N), dtype=jnp.float32).at[row, col].add(ew_const)
    A = A + jnp.eye(N, dtype=jnp.float32)
    A = jnp.minimum(A, 1.0)
    A2 = A @ A
    diag = jnp.arange(N)
    A_nd = A.at[diag, diag].set(0.0)
    A2_nd = A2.at[diag, diag].set(0.0)
    Dg = jnp.sum(A_nd != 0, axis=1)
    Dr = jnp.sum(A2_nd != 0, axis=1)
    Dgf = Dg.astype(jnp.float32)
    d_thres = jnp.mean(Dgf) + 2.0 * jnp.std(Dgf)
    cond1 = Dgf > d_thres
    two_d = jnp.round(2.0 * Dgf).astype(Dg.dtype)
    cond2 = (~cond1) & (two_d > Dr)
    cond3 = (~cond1) & (~cond2)
    nbr = A2_nd != 0
    sq = jnp.sum(X * X, axis=1)
    d2 = sq[:, None] + sq[None, :] - 2.0 * (X @ X.T)
    dist_m = jnp.where(nbr, d2, -jnp.inf)
    order = jnp.argsort(-dist_m, axis=1)
    ranks = jnp.argsort(order, axis=1)
    nz_sel = Dr - two_d
    k_eff = jnp.where(nz_sel == 0, Dr, nz_sel)
    mask3 = cond3[:, None] & nbr & (ranks < k_eff[:, None])
    mask1 = cond1[:, None] & (A_nd == 0)
    full_mask = mask1 | mask3
    adj_k = (A2_nd - A_nd).at[diag, diag].set(0.0)
    adj_k = jnp.where(full_mask, 0.0, adj_k)
    vals = adj_k[row, col]
    return _combine(theta_1, theta_2, edge_weight, vals)


# speedup vs baseline: 3.0613x; 3.0602x over previous
import functools

import jax
import jax.numpy as jnp
from jax.experimental import pallas as pl
from jax.experimental.pallas import tpu as pltpu

N = 4096
INT32_MIN = -(2**31)
INT32_MAX = 2**31 - 1


def _clipped_block(raw, gi, gj, bm, bk):
    rows = gi * bm + jax.lax.broadcasted_iota(jnp.int32, (bm, bk), 0)
    cols = gj * bk + jax.lax.broadcasted_iota(jnp.int32, (bm, bk), 1)
    eye = jnp.where(rows == cols, 1.0, 0.0)
    return jnp.minimum(raw + eye, 1.0)


def _a2_body(araw_l_ref, araw_r_ref, a2_ref, dg_ref, acc_ref, *, bm, bn, bk, nk):
    i = pl.program_id(0)
    j = pl.program_id(1)
    k = pl.program_id(2)
    a = _clipped_block(araw_l_ref[...], i, k, bm, bk)
    b = _clipped_block(araw_r_ref[...], k, j, bk, bn)

    @pl.when(k == 0)
    def _():
        acc_ref[...] = jnp.zeros_like(acc_ref)

    acc_ref[...] += jnp.dot(a, b, preferred_element_type=jnp.float32)

    @pl.when(k == nk - 1)
    def _():
        a2_ref[...] = acc_ref[...]

    @pl.when(j == 0)
    def _():
        rows = i * bm + jax.lax.broadcasted_iota(jnp.int32, (bm, bk), 0)
        cols = k * bk + jax.lax.broadcasted_iota(jnp.int32, (bm, bk), 1)
        cnt = jnp.sum(((a > 0.0) & (rows != cols)).astype(jnp.float32), axis=1)

        @pl.when(k == 0)
        def _():
            dg_ref[...] = cnt

        @pl.when(k > 0)
        def _():
            dg_ref[...] += cnt


def _a2_matmul(araw):
    bm = bn = bk = 512
    nk = N // bk
    grid = (N // bm, N // bn, nk)
    return pl.pallas_call(
        functools.partial(_a2_body, bm=bm, bn=bn, bk=bk, nk=nk),
        grid=grid,
        in_specs=[
            pl.BlockSpec((bm, bk), lambda i, j, k: (i, k)),
            pl.BlockSpec((bk, bn), lambda i, j, k: (k, j)),
        ],
        out_specs=[
            pl.BlockSpec((bm, bn), lambda i, j, k: (i, j)),
            pl.BlockSpec((bm,), lambda i, j, k: (i,)),
        ],
        out_shape=[
            jax.ShapeDtypeStruct((N, N), jnp.float32),
            jax.ShapeDtypeStruct((N,), jnp.float32),
        ],
        scratch_shapes=[pltpu.VMEM((bm, bn), jnp.float32)],
    )(araw, araw)


def _sortable(d2):
    i = jax.lax.bitcast_convert_type(d2, jnp.int32)
    return i ^ ((i >> 31) & jnp.int32(0x7FFFFFFF))


def _keys_body(xi_ref, xj_ref, out_ref):
    xi = xi_ref[...]
    xj = xj_ref[...]
    sqi = jnp.sum(xi * xi, axis=1)
    sqj = jnp.sum(xj * xj, axis=1)
    g = jax.lax.dot_general(xi, xj, (((1,), (1,)), ((), ())),
                            preferred_element_type=jnp.float32)
    d2 = sqi[:, None] + sqj[None, :] - 2.0 * g
    out_ref[...] = _sortable(d2)


def _dist_keys(x):
    br = 256
    grid = (N // br, N // br)
    d = x.shape[1]
    return pl.pallas_call(
        _keys_body,
        grid=grid,
        in_specs=[
            pl.BlockSpec((br, d), lambda i, j: (i, 0)),
            pl.BlockSpec((br, d), lambda i, j: (j, 0)),
        ],
        out_specs=pl.BlockSpec((br, br), lambda i, j: (i, j)),
        out_shape=jax.ShapeDtypeStruct((N, N), jnp.int32),
    )(x, x)


def _thresh_body(dg_ref, a2_ref, keys_ref, tkey_ref, cls_ref, *, br):
    i = pl.program_id(0)
    a2 = a2_ref[...]
    rows = i * br + jax.lax.broadcasted_iota(jnp.int32, (br, N), 0)
    cols = jax.lax.broadcasted_iota(jnp.int32, (br, N), 1)
    nbr = (a2 > 0.0) & (rows != cols)
    dr = jnp.sum(nbr.astype(jnp.float32), axis=1).astype(jnp.int32)

    dgf_all = dg_ref[...]
    mean = jnp.mean(dgf_all)
    std = jnp.sqrt(jnp.mean((dgf_all - mean) ** 2))
    d_thres = mean + 2.0 * std

    dgf = dg_ref[pl.ds(i * br, br)]
    two_d = jnp.round(2.0 * dgf).astype(jnp.int32)
    cond1 = dgf > d_thres
    cond2 = (~cond1) & (two_d > dr)
    cond3 = (~cond1) & (~cond2)
    cls = jnp.where(cond1, 1, jnp.where(cond2, 2, 3)).astype(jnp.int32)

    kq = jnp.where(cond3 & (two_d < dr) & (two_d > 0), two_d, 0)
    k_idx = kq - 1

    keysm = jnp.where(nbr, keys_ref[...], INT32_MAX)

    def body(t, lo):
        b = 31 - t
        mid = lo + (jnp.int32(1) << b)
        cnt = jnp.sum((keysm < mid[:, None]).astype(jnp.int32), axis=1)
        return jnp.where(cnt <= k_idx, mid, lo)

    lo = jax.lax.fori_loop(0, 32, body, jnp.full((br,), INT32_MIN, jnp.int32))
    tkey_ref[...] = jnp.where(kq > 0, lo, INT32_MIN)
    cls_ref[...] = cls


def _thresholds(dgf, a2, keys):
    br = 128
    grid = (N // br,)
    return pl.pallas_call(
        functools.partial(_thresh_body, br=br),
        grid=grid,
        in_specs=[
            pl.BlockSpec((N,), lambda i: (0,)),
            pl.BlockSpec((br, N), lambda i: (i, 0)),
            pl.BlockSpec((br, N), lambda i: (i, 0)),
        ],
        out_specs=[
            pl.BlockSpec((br,), lambda i: (i,)),
            pl.BlockSpec((br,), lambda i: (i,)),
        ],
        out_shape=[
            jax.ShapeDtypeStruct((N,), jnp.int32),
            jax.ShapeDtypeStruct((N,), jnp.int32),
        ],
    )(dgf, a2, keys)


def _combine_body(th_ref, ew_ref, r_ref, c_ref, araw_ref, a2_ref, key_ref,
                  tk_ref, cls_ref, o_ref):
    t1 = th_ref[0]
    t2 = th_ref[1]
    r = r_ref[...]
    c = c_ref[...]
    a = jnp.minimum(araw_ref[...], 1.0)
    a2 = a2_ref[...]
    base = a2 - a
    cls = cls_ref[...]
    val1 = jnp.where(a > 0.0, base, 0.0)
    val3 = jnp.where((a2 > 0.0) & (key_ref[...] > tk_ref[...]), 0.0, base)
    val = jnp.where(cls == 1, val1, jnp.where(cls == 2, base, val3))
    val = jnp.where(r == c, 0.0, val)
    o_ref[...] = jnp.maximum(t1 * ew_ref[...] + t2 * val, 0.0)


def _combine(theta_1, theta_2, ew, r, c, araw_e, a2_e, key_e, tk_e, cls_e):
    E = ew.shape[0]
    R = E // 128
    th = jnp.concatenate([theta_1, theta_2])
    rs = lambda a: a.reshape(R, 128)
    out = pl.pallas_call(
        _combine_body,
        in_specs=[pl.BlockSpec(memory_space=pltpu.SMEM)]
        + [pl.BlockSpec((R, 128), lambda: (0, 0))] * 8,
        out_specs=pl.BlockSpec((R, 128), lambda: (0, 0)),
        out_shape=jax.ShapeDtypeStruct((R, 128), jnp.float32),
    )(th, rs(ew), rs(r), rs(c), rs(araw_e), rs(a2_e), rs(key_e), rs(tk_e),
      rs(cls_e))
    return out.reshape(E)


def kernel(edge_index, edge_weight, features, theta_1, theta_2):
    row, col = edge_index[0], edge_index[1]
    ew_const = jax.lax.stop_gradient(edge_weight)
    x = jax.lax.stop_gradient(features)

    araw = jnp.zeros((N, N), jnp.float32).at[row, col].add(ew_const)
    a2, dgf = _a2_matmul(araw)
    keys = _dist_keys(x)
    tkey, cls = _thresholds(dgf, a2, keys)

    flat = row * N + col
    araw_e = araw.reshape(-1)[flat]
    a2_e = a2.reshape(-1)[flat]
    key_e = keys.reshape(-1)[flat]
    tk_e = tkey[row]
    cls_e = cls[row]
    return _combine(theta_1, theta_2, edge_weight, row.astype(jnp.int32),
                    col.astype(jnp.int32), araw_e, a2_e, key_e, tk_e, cls_e)


# speedup vs baseline: 3.0931x; 1.0104x over previous
import functools

import jax
import jax.numpy as jnp
from jax.experimental import pallas as pl
from jax.experimental.pallas import tpu as pltpu

N = 4096
INT32_MIN = -(2**31)
INT32_MAX = 2**31 - 1


def _clip_body(araw_ref, a_ref, dg_ref, *, bm):
    i = pl.program_id(0)
    rows = i * bm + jax.lax.broadcasted_iota(jnp.int32, (bm, N), 0)
    cols = jax.lax.broadcasted_iota(jnp.int32, (bm, N), 1)
    ond = rows == cols
    a = jnp.minimum(araw_ref[...] + jnp.where(ond, 1.0, 0.0), 1.0)
    a_ref[...] = a
    dg_ref[...] = jnp.sum(((a > 0.0) & ~ond).astype(jnp.float32), axis=1)


def _clip_a(araw):
    bm = 512
    return pl.pallas_call(
        functools.partial(_clip_body, bm=bm),
        grid=(N // bm,),
        in_specs=[pl.BlockSpec((bm, N), lambda i: (i, 0))],
        out_specs=[
            pl.BlockSpec((bm, N), lambda i: (i, 0)),
            pl.BlockSpec((bm,), lambda i: (i,)),
        ],
        out_shape=[
            jax.ShapeDtypeStruct((N, N), jnp.float32),
            jax.ShapeDtypeStruct((N,), jnp.float32),
        ],
    )(araw)


def _a2_body(a_l_ref, a_r_ref, a2_ref, acc_ref, *, nk):
    k = pl.program_id(2)

    @pl.when(k == 0)
    def _():
        acc_ref[...] = jnp.zeros_like(acc_ref)

    acc_ref[...] += jnp.dot(a_l_ref[...], a_r_ref[...],
                            preferred_element_type=jnp.float32)

    @pl.when(k == nk - 1)
    def _():
        a2_ref[...] = acc_ref[...]


def _a2_matmul(a):
    bm = bn = bk = 512
    nk = N // bk
    grid = (N // bm, N // bn, nk)
    return pl.pallas_call(
        functools.partial(_a2_body, nk=nk),
        grid=grid,
        in_specs=[
            pl.BlockSpec((bm, bk), lambda i, j, k: (i, k)),
            pl.BlockSpec((bk, bn), lambda i, j, k: (k, j)),
        ],
        out_specs=pl.BlockSpec((bm, bn), lambda i, j, k: (i, j)),
        out_shape=jax.ShapeDtypeStruct((N, N), jnp.float32),
        scratch_shapes=[pltpu.VMEM((bm, bn), jnp.float32)],
    )(a, a)


def _sortable(d2):
    i = jax.lax.bitcast_convert_type(d2, jnp.int32)
    return i ^ ((i >> 31) & jnp.int32(0x7FFFFFFF))


def _keys_body(xi_ref, xj_ref, out_ref):
    xi = xi_ref[...]
    xj = xj_ref[...]
    sqi = jnp.sum(xi * xi, axis=1)
    sqj = jnp.sum(xj * xj, axis=1)
    g = jax.lax.dot_general(xi, xj, (((1,), (1,)), ((), ())),
                            preferred_element_type=jnp.float32)
    d2 = sqi[:, None] + sqj[None, :] - 2.0 * g
    out_ref[...] = _sortable(d2)


def _dist_keys(x):
    br = 256
    grid = (N // br, N // br)
    d = x.shape[1]
    return pl.pallas_call(
        _keys_body,
        grid=grid,
        in_specs=[
            pl.BlockSpec((br, d), lambda i, j: (i, 0)),
            pl.BlockSpec((br, d), lambda i, j: (j, 0)),
        ],
        out_specs=pl.BlockSpec((br, br), lambda i, j: (i, j)),
        out_shape=jax.ShapeDtypeStruct((N, N), jnp.int32),
    )(x, x)


def _thresh_body(dg_ref, a2_ref, keys_ref, tkey_ref, cls_ref, *, br):
    i = pl.program_id(0)
    a2 = a2_ref[...]
    rows = i * br + jax.lax.broadcasted_iota(jnp.int32, (br, N), 0)
    cols = jax.lax.broadcasted_iota(jnp.int32, (br, N), 1)
    nbr = (a2 > 0.0) & (rows != cols)
    dr = jnp.sum(nbr.astype(jnp.float32), axis=1).astype(jnp.int32)

    dgf_all = dg_ref[...]
    mean = jnp.mean(dgf_all)
    std = jnp.sqrt(jnp.mean((dgf_all - mean) ** 2))
    d_thres = mean + 2.0 * std

    dgf = dg_ref[pl.ds(i * br, br)]
    two_d = jnp.round(2.0 * dgf).astype(jnp.int32)
    cond1 = dgf > d_thres
    cond2 = (~cond1) & (two_d > dr)
    cond3 = (~cond1) & (~cond2)
    cls = jnp.where(cond1, 1, jnp.where(cond2, 2, 3)).astype(jnp.int32)

    kq = jnp.where(cond3 & (two_d < dr) & (two_d > 0), two_d, 0)
    k_idx = kq - 1

    keysm = jnp.where(nbr, keys_ref[...], INT32_MAX)

    def body(t, lo):
        b = 31 - t
        mid = lo + (jnp.int32(1) << b)
        cnt = jnp.sum((keysm < mid[:, None]).astype(jnp.int32), axis=1)
        return jnp.where(cnt <= k_idx, mid, lo)

    lo = jax.lax.fori_loop(0, 32, body, jnp.full((br,), INT32_MIN, jnp.int32))
    tkey_ref[...] = jnp.where(kq > 0, lo, INT32_MIN)
    cls_ref[...] = cls


def _thresholds(dgf, a2, keys):
    br = 128
    grid = (N // br,)
    return pl.pallas_call(
        functools.partial(_thresh_body, br=br),
        grid=grid,
        in_specs=[
            pl.BlockSpec((N,), lambda i: (0,)),
            pl.BlockSpec((br, N), lambda i: (i, 0)),
            pl.BlockSpec((br, N), lambda i: (i, 0)),
        ],
        out_specs=[
            pl.BlockSpec((br,), lambda i: (i,)),
            pl.BlockSpec((br,), lambda i: (i,)),
        ],
        out_shape=[
            jax.ShapeDtypeStruct((N,), jnp.int32),
            jax.ShapeDtypeStruct((N,), jnp.int32),
        ],
    )(dgf, a2, keys)


def _combine_body(th_ref, ew_ref, r_ref, c_ref, araw_ref, a2_ref, key_ref,
                  tk_ref, cls_ref, o_ref):
    t1 = th_ref[0]
    t2 = th_ref[1]
    r = r_ref[...]
    c = c_ref[...]
    a = araw_ref[...]
    a2 = a2_ref[...]
    base = a2 - a
    cls = cls_ref[...]
    val1 = jnp.where(a > 0.0, base, 0.0)
    val3 = jnp.where((a2 > 0.0) & (key_ref[...] > tk_ref[...]), 0.0, base)
    val = jnp.where(cls == 1, val1, jnp.where(cls == 2, base, val3))
    val = jnp.where(r == c, 0.0, val)
    o_ref[...] = jnp.maximum(t1 * ew_ref[...] + t2 * val, 0.0)


def _combine(theta_1, theta_2, ew, r, c, araw_e, a2_e, key_e, tk_e, cls_e):
    E = ew.shape[0]
    R = E // 128
    th = jnp.concatenate([theta_1, theta_2])
    rs = lambda a: a.reshape(R, 128)
    out = pl.pallas_call(
        _combine_body,
        in_specs=[pl.BlockSpec(memory_space=pltpu.SMEM)]
        + [pl.BlockSpec((R, 128), lambda: (0, 0))] * 8,
        out_specs=pl.BlockSpec((R, 128), lambda: (0, 0)),
        out_shape=jax.ShapeDtypeStruct((R, 128), jnp.float32),
    )(th, rs(ew), rs(r), rs(c), rs(araw_e), rs(a2_e), rs(key_e), rs(tk_e),
      rs(cls_e))
    return out.reshape(E)


def kernel(edge_index, edge_weight, features, theta_1, theta_2):
    row, col = edge_index[0], edge_index[1]
    ew_const = jax.lax.stop_gradient(edge_weight)
    x = jax.lax.stop_gradient(features)

    araw = jnp.zeros((N, N), jnp.float32).at[row, col].add(ew_const)
    a, dgf = _clip_a(araw)
    a2 = _a2_matmul(a)
    keys = _dist_keys(x)
    tkey, cls = _thresholds(dgf, a2, keys)

    flat = row * N + col
    araw_e = a.reshape(-1)[flat]
    a2_e = a2.reshape(-1)[flat]
    key_e = keys.reshape(-1)[flat]
    tk_e = tkey[row]
    cls_e = cls[row]
    return _combine(theta_1, theta_2, edge_weight, row.astype(jnp.int32),
                    col.astype(jnp.int32), araw_e, a2_e, key_e, tk_e, cls_e)


# speedup vs baseline: 3.2286x; 1.0438x over previous
import functools

import jax
import jax.numpy as jnp
from jax.experimental import pallas as pl
from jax.experimental.pallas import tpu as pltpu

N = 4096
INT32_MIN = -(2**31)
INT32_MAX = 2**31 - 1


def _clip_body(araw_ref, a_ref, dg_ref, *, bm):
    i = pl.program_id(0)
    rows = i * bm + jax.lax.broadcasted_iota(jnp.int32, (bm, N), 0)
    cols = jax.lax.broadcasted_iota(jnp.int32, (bm, N), 1)
    ond = rows == cols
    a = jnp.minimum(araw_ref[...] + jnp.where(ond, 1.0, 0.0), 1.0)
    a_ref[...] = a
    dg_ref[...] = jnp.sum(((a > 0.0) & ~ond).astype(jnp.float32), axis=1)


def _clip_a(araw):
    bm = 512
    return pl.pallas_call(
        functools.partial(_clip_body, bm=bm),
        grid=(N // bm,),
        in_specs=[pl.BlockSpec((bm, N), lambda i: (i, 0))],
        out_specs=[
            pl.BlockSpec((bm, N), lambda i: (i, 0)),
            pl.BlockSpec((bm,), lambda i: (i,)),
        ],
        out_shape=[
            jax.ShapeDtypeStruct((N, N), jnp.float32),
            jax.ShapeDtypeStruct((N,), jnp.float32),
        ],
    )(araw)


def _a2_body(a_l_ref, a_r_ref, a2_ref, acc_ref, *, nk):
    k = pl.program_id(2)

    @pl.when(k == 0)
    def _():
        acc_ref[...] = jnp.zeros_like(acc_ref)

    acc_ref[...] += jnp.dot(a_l_ref[...], a_r_ref[...],
                            preferred_element_type=jnp.float32)

    @pl.when(k == nk - 1)
    def _():
        a2_ref[...] = acc_ref[...]


def _a2_matmul(a):
    bm = bn = bk = 512
    nk = N // bk
    grid = (N // bm, N // bn, nk)
    return pl.pallas_call(
        functools.partial(_a2_body, nk=nk),
        grid=grid,
        in_specs=[
            pl.BlockSpec((bm, bk), lambda i, j, k: (i, k)),
            pl.BlockSpec((bk, bn), lambda i, j, k: (k, j)),
        ],
        out_specs=pl.BlockSpec((bm, bn), lambda i, j, k: (i, j)),
        out_shape=jax.ShapeDtypeStruct((N, N), jnp.float32),
        scratch_shapes=[pltpu.VMEM((bm, bn), jnp.float32)],
    )(a, a)


def _sortable(d2):
    i = jax.lax.bitcast_convert_type(d2, jnp.int32)
    return i ^ ((i >> 31) & jnp.int32(0x7FFFFFFF))


def _keys_body(xi_ref, xj_ref, out_ref):
    xi = xi_ref[...]
    xj = xj_ref[...]
    sqi = jnp.sum(xi * xi, axis=1)
    sqj = jnp.sum(xj * xj, axis=1)
    g = jax.lax.dot_general(xi, xj, (((1,), (1,)), ((), ())),
                            preferred_element_type=jnp.float32)
    d2 = sqi[:, None] + sqj[None, :] - 2.0 * g
    out_ref[...] = _sortable(d2)


def _dist_keys(x):
    br = 256
    grid = (N // br, N // br)
    d = x.shape[1]
    return pl.pallas_call(
        _keys_body,
        grid=grid,
        in_specs=[
            pl.BlockSpec((br, d), lambda i, j: (i, 0)),
            pl.BlockSpec((br, d), lambda i, j: (j, 0)),
        ],
        out_specs=pl.BlockSpec((br, br), lambda i, j: (i, j)),
        out_shape=jax.ShapeDtypeStruct((N, N), jnp.int32),
    )(x, x)


def _thresh_body(dg_ref, a2_ref, keys_ref, tkey_ref, cls_ref, *, br):
    i = pl.program_id(0)
    a2 = a2_ref[...]
    rows = i * br + jax.lax.broadcasted_iota(jnp.int32, (br, N), 0)
    cols = jax.lax.broadcasted_iota(jnp.int32, (br, N), 1)
    nbr = (a2 > 0.0) & (rows != cols)
    dr = jnp.sum(nbr.astype(jnp.float32), axis=1).astype(jnp.int32)

    dgf_all = dg_ref[...]
    mean = jnp.mean(dgf_all)
    std = jnp.sqrt(jnp.mean((dgf_all - mean) ** 2))
    d_thres = mean + 2.0 * std

    dgf = dg_ref[pl.ds(i * br, br)]
    two_d = jnp.round(2.0 * dgf).astype(jnp.int32)
    cond1 = dgf > d_thres
    cond2 = (~cond1) & (two_d > dr)
    cond3 = (~cond1) & (~cond2)
    cls = jnp.where(cond1, 1, jnp.where(cond2, 2, 3)).astype(jnp.int32)

    kq = jnp.where(cond3 & (two_d < dr) & (two_d > 0), two_d, 0)
    k_idx = kq - 1

    keysm = jnp.where(nbr, keys_ref[...], INT32_MAX)

    def body(t, lo):
        b = 31 - t
        mid = lo + (jnp.int32(1) << b)
        cnt = jnp.sum((keysm < mid[:, None]).astype(jnp.int32), axis=1)
        return jnp.where(cnt <= k_idx, mid, lo)

    lo = jax.lax.fori_loop(0, 32, body, jnp.full((br,), INT32_MIN, jnp.int32))
    tkey_ref[...] = jnp.where(kq > 0, lo, INT32_MIN)
    cls_ref[...] = cls


def _thresholds(dgf, a2, keys):
    br = 128
    grid = (N // br,)
    return pl.pallas_call(
        functools.partial(_thresh_body, br=br),
        grid=grid,
        in_specs=[
            pl.BlockSpec((N,), lambda i: (0,)),
            pl.BlockSpec((br, N), lambda i: (i, 0)),
            pl.BlockSpec((br, N), lambda i: (i, 0)),
        ],
        out_specs=[
            pl.BlockSpec((br,), lambda i: (i,)),
            pl.BlockSpec((br,), lambda i: (i,)),
        ],
        out_shape=[
            jax.ShapeDtypeStruct((N,), jnp.int32),
            jax.ShapeDtypeStruct((N,), jnp.int32),
        ],
    )(dgf, a2, keys)


def _combine_body(th_ref, ew_ref, r_ref, c_ref, araw_ref, a2_ref, key_ref,
                  tk_ref, cls_ref, o_ref):
    t1 = th_ref[0]
    t2 = th_ref[1]
    r = r_ref[...]
    c = c_ref[...]
    a = araw_ref[...]
    a2 = a2_ref[...]
    base = a2 - a
    cls = cls_ref[...]
    val1 = jnp.where(a > 0.0, base, 0.0)
    val3 = jnp.where((a2 > 0.0) & (key_ref[...] > tk_ref[...]), 0.0, base)
    val = jnp.where(cls == 1, val1, jnp.where(cls == 2, base, val3))
    val = jnp.where(r == c, 0.0, val)
    o_ref[...] = jnp.maximum(t1 * ew_ref[...] + t2 * val, 0.0)


def _combine(theta_1, theta_2, ew, r, c, araw_e, a2_e, key_e, tk_e, cls_e):
    E = ew.shape[0]
    R = E // 128
    th = jnp.concatenate([theta_1, theta_2])
    rs = lambda a: a.reshape(R, 128)
    out = pl.pallas_call(
        _combine_body,
        in_specs=[pl.BlockSpec(memory_space=pltpu.SMEM)]
        + [pl.BlockSpec((R, 128), lambda: (0, 0))] * 8,
        out_specs=pl.BlockSpec((R, 128), lambda: (0, 0)),
        out_shape=jax.ShapeDtypeStruct((R, 128), jnp.float32),
    )(th, rs(ew), rs(r), rs(c), rs(araw_e), rs(a2_e), rs(key_e), rs(tk_e),
      rs(cls_e))
    return out.reshape(E)


def kernel(edge_index, edge_weight, features, theta_1, theta_2):
    row, col = edge_index[0], edge_index[1]
    ew_const = jax.lax.stop_gradient(edge_weight)
    x = jax.lax.stop_gradient(features)

    araw = jnp.zeros((N, N), jnp.float32).at[row, col].add(ew_const)
    a, dgf = _clip_a(araw)
    a2 = _a2_matmul(a)
    keys = _dist_keys(x)
    tkey, cls = _thresholds(dgf, a2, keys)

    araw_e = a[row, col]
    a2_e = a2[row, col]
    key_e = keys[row, col]
    tk_e = tkey[row]
    cls_e = cls[row]
    return _combine(theta_1, theta_2, edge_weight, row.astype(jnp.int32),
                    col.astype(jnp.int32), araw_e, a2_e, key_e, tk_e, cls_e)


# speedup vs baseline: 5.9415x; 1.8403x over previous
import functools

import jax
import jax.numpy as jnp
from jax import lax
from jax.experimental import pallas as pl
from jax.experimental.pallas import tpu as pltpu
from jax.experimental.pallas import tpu_sc as plsc

N = 4096
INT32_MIN = -(2**31)
INT32_MAX = 2**31 - 1


def _clip_body(araw_ref, a_ref, dg_ref, *, bm):
    i = pl.program_id(0)
    rows = i * bm + jax.lax.broadcasted_iota(jnp.int32, (bm, N), 0)
    cols = jax.lax.broadcasted_iota(jnp.int32, (bm, N), 1)
    ond = rows == cols
    a = jnp.minimum(araw_ref[...] + jnp.where(ond, 1.0, 0.0), 1.0)
    a_ref[...] = a
    dg_ref[...] = jnp.sum(((a > 0.0) & ~ond).astype(jnp.float32), axis=1)


def _clip_a(araw):
    bm = 512
    return pl.pallas_call(
        functools.partial(_clip_body, bm=bm),
        grid=(N // bm,),
        in_specs=[pl.BlockSpec((bm, N), lambda i: (i, 0))],
        out_specs=[
            pl.BlockSpec((bm, N), lambda i: (i, 0)),
            pl.BlockSpec((bm,), lambda i: (i,)),
        ],
        out_shape=[
            jax.ShapeDtypeStruct((N, N), jnp.float32),
            jax.ShapeDtypeStruct((N,), jnp.float32),
        ],
    )(araw)


def _a2_body(a_l_ref, a_r_ref, a2_ref, acc_ref, *, nk):
    k = pl.program_id(2)

    @pl.when(k == 0)
    def _():
        acc_ref[...] = jnp.zeros_like(acc_ref)

    acc_ref[...] += jnp.dot(a_l_ref[...], a_r_ref[...],
                            preferred_element_type=jnp.float32)

    @pl.when(k == nk - 1)
    def _():
        a2_ref[...] = acc_ref[...]


def _a2_matmul(a):
    bm = bn = bk = 512
    nk = N // bk
    grid = (N // bm, N // bn, nk)
    return pl.pallas_call(
        functools.partial(_a2_body, nk=nk),
        grid=grid,
        in_specs=[
            pl.BlockSpec((bm, bk), lambda i, j, k: (i, k)),
            pl.BlockSpec((bk, bn), lambda i, j, k: (k, j)),
        ],
        out_specs=pl.BlockSpec((bm, bn), lambda i, j, k: (i, j)),
        out_shape=jax.ShapeDtypeStruct((N, N), jnp.float32),
        scratch_shapes=[pltpu.VMEM((bm, bn), jnp.float32)],
    )(a, a)


def _sortable(d2):
    i = jax.lax.bitcast_convert_type(d2, jnp.int32)
    return i ^ ((i >> 31) & jnp.int32(0x7FFFFFFF))


def _keys_body(xi_ref, xj_ref, out_ref):
    xi = xi_ref[...]
    xj = xj_ref[...]
    sqi = jnp.sum(xi * xi, axis=1)
    sqj = jnp.sum(xj * xj, axis=1)
    g = jax.lax.dot_general(xi, xj, (((1,), (1,)), ((), ())),
                            preferred_element_type=jnp.float32)
    d2 = sqi[:, None] + sqj[None, :] - 2.0 * g
    out_ref[...] = _sortable(d2)


def _dist_keys(x):
    br = 256
    grid = (N // br, N // br)
    d = x.shape[1]
    return pl.pallas_call(
        _keys_body,
        grid=grid,
        in_specs=[
            pl.BlockSpec((br, d), lambda i, j: (i, 0)),
            pl.BlockSpec((br, d), lambda i, j: (j, 0)),
        ],
        out_specs=pl.BlockSpec((br, br), lambda i, j: (i, j)),
        out_shape=jax.ShapeDtypeStruct((N, N), jnp.int32),
    )(x, x)


def _thresh_body(dg_ref, a2_ref, keys_ref, tkey_ref, cls_ref, *, br):
    i = pl.program_id(0)
    a2 = a2_ref[...]
    rows = i * br + jax.lax.broadcasted_iota(jnp.int32, (br, N), 0)
    cols = jax.lax.broadcasted_iota(jnp.int32, (br, N), 1)
    nbr = (a2 > 0.0) & (rows != cols)
    dr = jnp.sum(nbr.astype(jnp.float32), axis=1).astype(jnp.int32)

    dgf_all = dg_ref[...]
    mean = jnp.mean(dgf_all)
    std = jnp.sqrt(jnp.mean((dgf_all - mean) ** 2))
    d_thres = mean + 2.0 * std

    dgf = dg_ref[pl.ds(i * br, br)]
    two_d = jnp.round(2.0 * dgf).astype(jnp.int32)
    cond1 = dgf > d_thres
    cond2 = (~cond1) & (two_d > dr)
    cond3 = (~cond1) & (~cond2)
    cls = jnp.where(cond1, 1, jnp.where(cond2, 2, 3)).astype(jnp.int32)

    kq = jnp.where(cond3 & (two_d < dr) & (two_d > 0), two_d, 0)
    k_idx = kq - 1

    keysm = jnp.where(nbr, keys_ref[...], INT32_MAX)

    def body(t, lo):
        b = 31 - t
        mid = lo + (jnp.int32(1) << b)
        cnt = jnp.sum((keysm < mid[:, None]).astype(jnp.int32), axis=1)
        return jnp.where(cnt <= k_idx, mid, lo)

    lo = jax.lax.fori_loop(0, 32, body, jnp.full((br,), INT32_MIN, jnp.int32))
    tkey_ref[...] = jnp.where(kq > 0, lo, INT32_MIN)
    cls_ref[...] = cls


def _thresholds(dgf, a2, keys):
    br = 128
    grid = (N // br,)
    return pl.pallas_call(
        functools.partial(_thresh_body, br=br),
        grid=grid,
        in_specs=[
            pl.BlockSpec((N,), lambda i: (0,)),
            pl.BlockSpec((br, N), lambda i: (i, 0)),
            pl.BlockSpec((br, N), lambda i: (i, 0)),
        ],
        out_specs=[
            pl.BlockSpec((br,), lambda i: (i,)),
            pl.BlockSpec((br,), lambda i: (i,)),
        ],
        out_shape=[
            jax.ShapeDtypeStruct((N,), jnp.int32),
            jax.ShapeDtypeStruct((N,), jnp.int32),
        ],
    )(dgf, a2, keys)


def _edge_combine(ridx, cidx, ew, arawv, a2v, keyv, tkey, cls, th1v, th2v):
    E = ew.shape[0]
    NW = 32
    CH = E // NW
    mesh = plsc.VectorSubcoreMesh(core_axis_name="c", subcore_axis_name="s")

    @functools.partial(
        pl.kernel,
        out_type=jax.ShapeDtypeStruct((E,), jnp.float32),
        mesh=mesh,
        scratch_types=[
            pltpu.VMEM((CH,), jnp.int32),
            pltpu.VMEM((CH,), jnp.int32),
            pltpu.VMEM((CH,), jnp.float32),
            pltpu.VMEM((CH,), jnp.int32),
            pltpu.VMEM((CH,), jnp.int32),
            pltpu.VMEM((CH,), jnp.int32),
            pltpu.VMEM((CH,), jnp.float32),
            pltpu.VMEM((CH,), jnp.float32),
            pltpu.VMEM((CH,), jnp.int32),
            pltpu.VMEM((CH,), jnp.float32),
            pltpu.VMEM((16,), jnp.float32),
            pltpu.VMEM((16,), jnp.float32),
            pltpu.SemaphoreType.DMA,
        ],
    )
    def body(r_hbm, c_hbm, w_hbm, araw_hbm, a2_hbm, key_hbm, tk_hbm, cls_hbm,
             th1_hbm, th2_hbm, out_hbm, rbuf, cbuf, wbuf, fbuf, tkbuf, clsbuf,
             gab, g2b, gkb, obuf, t1b, t2b, sem):
        wid = lax.axis_index("s") * 2 + lax.axis_index("c")
        base = wid * CH
        pltpu.sync_copy(r_hbm.at[pl.ds(base, CH)], rbuf)
        pltpu.sync_copy(c_hbm.at[pl.ds(base, CH)], cbuf)
        pltpu.sync_copy(w_hbm.at[pl.ds(base, CH)], wbuf)
        pltpu.sync_copy(th1_hbm, t1b)
        pltpu.sync_copy(th2_hbm, t2b)

        def idx_body(i, carry):
            rv = rbuf[pl.ds(i * 16, 16)]
            cv = cbuf[pl.ds(i * 16, 16)]
            fbuf[pl.ds(i * 16, 16)] = rv * 4096 + cv
            return carry

        lax.fori_loop(0, CH // 16, idx_body, 0)

        descs = []
        for j in range(CH // 128):
            sl = pl.ds(j * 128, 128)
            descs.append(pltpu.async_copy(
                tk_hbm.at[rbuf.at[sl]], tkbuf.at[sl], sem))
            descs.append(pltpu.async_copy(
                cls_hbm.at[rbuf.at[sl]], clsbuf.at[sl], sem))
            descs.append(pltpu.async_copy(
                araw_hbm.at[fbuf.at[sl]], gab.at[sl], sem))
            descs.append(pltpu.async_copy(
                a2_hbm.at[fbuf.at[sl]], g2b.at[sl], sem))
            descs.append(pltpu.async_copy(
                key_hbm.at[fbuf.at[sl]], gkb.at[sl], sem))
        for d in descs:
            d.wait()

        t1 = t1b[...]
        t2 = t2b[...]

        def ext_body(v, carry):
            p = v * 16
            rv = rbuf[pl.ds(p, 16)]
            cv = cbuf[pl.ds(p, 16)]
            arawe = gab[pl.ds(p, 16)]
            a2e = g2b[pl.ds(p, 16)]
            keye = gkb[pl.ds(p, 16)]
            tke = tkbuf[pl.ds(p, 16)]
            clse = clsbuf[pl.ds(p, 16)]
            wv = wbuf[pl.ds(p, 16)]
            ae = jnp.minimum(arawe, 1.0)
            bv = a2e - ae
            val1 = jnp.where(ae > 0.0, bv, 0.0)
            val3 = jnp.where((a2e > 0.0) & (keye > tke), 0.0, bv)
            val = jnp.where(clse == 1, val1,
                            jnp.where(clse == 2, bv, val3))
            val = jnp.where(rv == cv, 0.0, val)
            obuf[pl.ds(p, 16)] = jnp.maximum(t1 * wv + t2 * val, 0.0)
            return carry

        lax.fori_loop(0, CH // 16, ext_body, 0)

        pltpu.sync_copy(obuf, out_hbm.at[pl.ds(base, CH)])

    return body(ridx, cidx, ew, arawv, a2v, keyv, tkey, cls, th1v, th2v)


def kernel(edge_index, edge_weight, features, theta_1, theta_2):
    row, col = edge_index[0], edge_index[1]
    ew_const = jax.lax.stop_gradient(edge_weight)
    x = jax.lax.stop_gradient(features)

    araw = jnp.zeros((N, N), jnp.float32).at[row, col].add(ew_const)
    a, dgf = _clip_a(araw)
    a2 = _a2_matmul(a)
    keys = _dist_keys(x)
    tkey, cls = _thresholds(dgf, a2, keys)

    th1v = jnp.broadcast_to(theta_1, (16,))
    th2v = jnp.broadcast_to(theta_2, (16,))
    return _edge_combine(row, col, edge_weight,
                         araw.reshape(N * N),
                         a2.reshape(N * N),
                         keys.reshape(N * N),
                         tkey, cls, th1v, th2v)


# speedup vs baseline: 7.4182x; 1.2485x over previous
import functools

import jax
import jax.numpy as jnp
from jax import lax
from jax.experimental import pallas as pl
from jax.experimental.pallas import tpu as pltpu
from jax.experimental.pallas import tpu_sc as plsc

N = 4096
INT32_MIN = -(2**31)
INT32_MAX = 2**31 - 1


def _clip_body(araw_ref, a_ref, dg_ref, *, bm):
    i = pl.program_id(0)
    rows = i * bm + jax.lax.broadcasted_iota(jnp.int32, (bm, N), 0)
    cols = jax.lax.broadcasted_iota(jnp.int32, (bm, N), 1)
    ond = rows == cols
    a = jnp.minimum(araw_ref[...] + jnp.where(ond, 1.0, 0.0), 1.0)
    a_ref[...] = a
    dg_ref[...] = jnp.sum(((a > 0.0) & ~ond).astype(jnp.float32), axis=1)


def _clip_a(araw):
    bm = 512
    return pl.pallas_call(
        functools.partial(_clip_body, bm=bm),
        grid=(N // bm,),
        in_specs=[pl.BlockSpec((bm, N), lambda i: (i, 0))],
        out_specs=[
            pl.BlockSpec((bm, N), lambda i: (i, 0)),
            pl.BlockSpec((bm,), lambda i: (i,)),
        ],
        out_shape=[
            jax.ShapeDtypeStruct((N, N), jnp.float32),
            jax.ShapeDtypeStruct((N,), jnp.float32),
        ],
    )(araw)


def _a2_body(a_l_ref, a_r_ref, a2_ref, acc_ref, *, nk):
    k = pl.program_id(2)

    @pl.when(k == 0)
    def _():
        acc_ref[...] = jnp.zeros_like(acc_ref)

    acc_ref[...] += jnp.dot(a_l_ref[...], a_r_ref[...],
                            preferred_element_type=jnp.float32)

    @pl.when(k == nk - 1)
    def _():
        a2_ref[...] = acc_ref[...]


def _a2_matmul(a):
    bm, bn, bk = 1024, 1024, 512
    nk = N // bk
    grid = (N // bm, N // bn, nk)
    return pl.pallas_call(
        functools.partial(_a2_body, nk=nk),
        grid=grid,
        in_specs=[
            pl.BlockSpec((bm, bk), lambda i, j, k: (i, k)),
            pl.BlockSpec((bk, bn), lambda i, j, k: (k, j)),
        ],
        out_specs=pl.BlockSpec((bm, bn), lambda i, j, k: (i, j)),
        out_shape=jax.ShapeDtypeStruct((N, N), jnp.float32),
        scratch_shapes=[pltpu.VMEM((bm, bn), jnp.float32)],
    )(a, a)


def _sortable(d2):
    i = jax.lax.bitcast_convert_type(d2, jnp.int32)
    return i ^ ((i >> 31) & jnp.int32(0x7FFFFFFF))


def _keys_body(xi_ref, xj_ref, out_ref):
    xi = xi_ref[...]
    xj = xj_ref[...]
    sqi = jnp.sum(xi * xi, axis=1)
    sqj = jnp.sum(xj * xj, axis=1)
    g = jax.lax.dot_general(xi, xj, (((1,), (1,)), ((), ())),
                            preferred_element_type=jnp.float32)
    d2 = sqi[:, None] + sqj[None, :] - 2.0 * g
    out_ref[...] = _sortable(d2)


def _dist_keys(x):
    br = 256
    grid = (N // br, N // br)
    d = x.shape[1]
    return pl.pallas_call(
        _keys_body,
        grid=grid,
        in_specs=[
            pl.BlockSpec((br, d), lambda i, j: (i, 0)),
            pl.BlockSpec((br, d), lambda i, j: (j, 0)),
        ],
        out_specs=pl.BlockSpec((br, br), lambda i, j: (i, j)),
        out_shape=jax.ShapeDtypeStruct((N, N), jnp.int32),
    )(x, x)


def _thresh_body(dg_ref, a2_ref, keys_ref, tkey_ref, cls_ref, *, br):
    i = pl.program_id(0)
    a2 = a2_ref[...]
    rows = i * br + jax.lax.broadcasted_iota(jnp.int32, (br, N), 0)
    cols = jax.lax.broadcasted_iota(jnp.int32, (br, N), 1)
    nbr = (a2 > 0.0) & (rows != cols)
    dr = jnp.sum(nbr.astype(jnp.float32), axis=1).astype(jnp.int32)

    dgf_all = dg_ref[...]
    mean = jnp.mean(dgf_all)
    std = jnp.sqrt(jnp.mean((dgf_all - mean) ** 2))
    d_thres = mean + 2.0 * std

    dgf = dg_ref[pl.ds(i * br, br)]
    two_d = jnp.round(2.0 * dgf).astype(jnp.int32)
    cond1 = dgf > d_thres
    cond2 = (~cond1) & (two_d > dr)
    cond3 = (~cond1) & (~cond2)
    cls = jnp.where(cond1, 1, jnp.where(cond2, 2, 3)).astype(jnp.int32)

    kq = jnp.where(cond3 & (two_d < dr) & (two_d > 0), two_d, 0)
    k_idx = kq - 1

    keysm = jnp.where(nbr, keys_ref[...], INT32_MAX)

    def body(t, lo):
        b = 31 - t
        mid = lo + (jnp.int32(1) << b)
        cnt = jnp.sum((keysm < mid[:, None]).astype(jnp.int32), axis=1)
        return jnp.where(cnt <= k_idx, mid, lo)

    lo = jax.lax.fori_loop(0, 32, body, jnp.full((br,), INT32_MIN, jnp.int32))
    tkey_ref[...] = jnp.where(kq > 0, lo, INT32_MIN)
    cls_ref[...] = cls


def _thresholds(dgf, a2, keys):
    br = 128
    grid = (N // br,)
    return pl.pallas_call(
        functools.partial(_thresh_body, br=br),
        grid=grid,
        in_specs=[
            pl.BlockSpec((N,), lambda i: (0,)),
            pl.BlockSpec((br, N), lambda i: (i, 0)),
            pl.BlockSpec((br, N), lambda i: (i, 0)),
        ],
        out_specs=[
            pl.BlockSpec((br,), lambda i: (i,)),
            pl.BlockSpec((br,), lambda i: (i,)),
        ],
        out_shape=[
            jax.ShapeDtypeStruct((N,), jnp.int32),
            jax.ShapeDtypeStruct((N,), jnp.int32),
        ],
    )(dgf, a2, keys)


def _edge_combine(ridx, cidx, ew, arawv, a2v, keyv, tkey, cls, th1v, th2v):
    E = ew.shape[0]
    NW = 32
    CH = E // NW
    mesh = plsc.VectorSubcoreMesh(core_axis_name="c", subcore_axis_name="s")

    @functools.partial(
        pl.kernel,
        out_type=jax.ShapeDtypeStruct((E,), jnp.float32),
        mesh=mesh,
        scratch_types=[
            pltpu.VMEM((CH,), jnp.int32),
            pltpu.VMEM((CH,), jnp.int32),
            pltpu.VMEM((CH,), jnp.float32),
            pltpu.VMEM((CH,), jnp.int32),
            pltpu.VMEM((CH,), jnp.int32),
            pltpu.VMEM((CH,), jnp.int32),
            pltpu.VMEM((CH,), jnp.float32),
            pltpu.VMEM((CH,), jnp.float32),
            pltpu.VMEM((CH,), jnp.int32),
            pltpu.VMEM((CH,), jnp.float32),
            pltpu.VMEM((16,), jnp.float32),
            pltpu.VMEM((16,), jnp.float32),
            pltpu.SemaphoreType.DMA,
        ],
    )
    def body(r_hbm, c_hbm, w_hbm, araw_hbm, a2_hbm, key_hbm, tk_hbm, cls_hbm,
             th1_hbm, th2_hbm, out_hbm, rbuf, cbuf, wbuf, fbuf, tkbuf, clsbuf,
             gab, g2b, gkb, obuf, t1b, t2b, sem):
        wid = lax.axis_index("s") * 2 + lax.axis_index("c")
        base = wid * CH
        pltpu.sync_copy(r_hbm.at[pl.ds(base, CH)], rbuf)
        pltpu.sync_copy(c_hbm.at[pl.ds(base, CH)], cbuf)
        pltpu.sync_copy(w_hbm.at[pl.ds(base, CH)], wbuf)
        pltpu.sync_copy(th1_hbm, t1b)
        pltpu.sync_copy(th2_hbm, t2b)

        def idx_body(i, carry):
            rv = rbuf[pl.ds(i * 16, 16)]
            cv = cbuf[pl.ds(i * 16, 16)]
            fbuf[pl.ds(i * 16, 16)] = rv * 4096 + cv
            return carry

        lax.fori_loop(0, CH // 16, idx_body, 0)

        descs = []
        for j in range(CH // 128):
            sl = pl.ds(j * 128, 128)
            descs.append(pltpu.async_copy(
                tk_hbm.at[rbuf.at[sl]], tkbuf.at[sl], sem))
            descs.append(pltpu.async_copy(
                cls_hbm.at[rbuf.at[sl]], clsbuf.at[sl], sem))
            descs.append(pltpu.async_copy(
                araw_hbm.at[fbuf.at[sl]], gab.at[sl], sem))
            descs.append(pltpu.async_copy(
                a2_hbm.at[fbuf.at[sl]], g2b.at[sl], sem))
            descs.append(pltpu.async_copy(
                key_hbm.at[fbuf.at[sl]], gkb.at[sl], sem))
        for d in descs:
            d.wait()

        t1 = t1b[...]
        t2 = t2b[...]

        def ext_body(v, carry):
            p = v * 16
            rv = rbuf[pl.ds(p, 16)]
            cv = cbuf[pl.ds(p, 16)]
            arawe = gab[pl.ds(p, 16)]
            a2e = g2b[pl.ds(p, 16)]
            keye = gkb[pl.ds(p, 16)]
            tke = tkbuf[pl.ds(p, 16)]
            clse = clsbuf[pl.ds(p, 16)]
            wv = wbuf[pl.ds(p, 16)]
            ae = jnp.minimum(arawe, 1.0)
            bv = a2e - ae
            val1 = jnp.where(ae > 0.0, bv, 0.0)
            val3 = jnp.where((a2e > 0.0) & (keye > tke), 0.0, bv)
            val = jnp.where(clse == 1, val1,
                            jnp.where(clse == 2, bv, val3))
            val = jnp.where(rv == cv, 0.0, val)
            obuf[pl.ds(p, 16)] = jnp.maximum(t1 * wv + t2 * val, 0.0)
            return carry

        lax.fori_loop(0, CH // 16, ext_body, 0)

        pltpu.sync_copy(obuf, out_hbm.at[pl.ds(base, CH)])

    return body(ridx, cidx, ew, arawv, a2v, keyv, tkey, cls, th1v, th2v)


def kernel(edge_index, edge_weight, features, theta_1, theta_2):
    row, col = edge_index[0], edge_index[1]
    ew_const = jax.lax.stop_gradient(edge_weight)
    x = jax.lax.stop_gradient(features)

    araw = jnp.zeros((N, N), jnp.float32).at[row, col].add(ew_const)
    a, dgf = _clip_a(araw)
    a2 = _a2_matmul(a)
    keys = _dist_keys(x)
    tkey, cls = _thresholds(dgf, a2, keys)

    th1v = jnp.broadcast_to(theta_1, (16,))
    th2v = jnp.broadcast_to(theta_2, (16,))
    return _edge_combine(row, col, edge_weight,
                         araw.reshape(N * N),
                         a2.reshape(N * N),
                         keys.reshape(N * N),
                         tkey, cls, th1v, th2v)


# speedup vs baseline: 8.2131x; 1.1072x over previous
import functools

import jax
import jax.numpy as jnp
from jax import lax
from jax.experimental import pallas as pl
from jax.experimental.pallas import tpu as pltpu
from jax.experimental.pallas import tpu_sc as plsc

N = 4096
INT32_MIN = -(2**31)
INT32_MAX = 2**31 - 1


def _clip_body(araw_ref, a_ref, dg_ref, *, bm):
    i = pl.program_id(0)
    rows = i * bm + jax.lax.broadcasted_iota(jnp.int32, (bm, N), 0)
    cols = jax.lax.broadcasted_iota(jnp.int32, (bm, N), 1)
    ond = rows == cols
    a = jnp.minimum(araw_ref[...] + jnp.where(ond, 1.0, 0.0), 1.0)
    a_ref[...] = a
    dg_ref[...] = jnp.sum(((a > 0.0) & ~ond).astype(jnp.float32), axis=1)


def _clip_a(araw):
    bm = 512
    return pl.pallas_call(
        functools.partial(_clip_body, bm=bm),
        grid=(N // bm,),
        in_specs=[pl.BlockSpec((bm, N), lambda i: (i, 0))],
        out_specs=[
            pl.BlockSpec((bm, N), lambda i: (i, 0)),
            pl.BlockSpec((bm,), lambda i: (i,)),
        ],
        out_shape=[
            jax.ShapeDtypeStruct((N, N), jnp.float32),
            jax.ShapeDtypeStruct((N,), jnp.float32),
        ],
    )(araw)


def _a2_body(a_l_ref, a_r_ref, a2_ref, acc_ref, *, nk):
    k = pl.program_id(2)

    @pl.when(k == 0)
    def _():
        acc_ref[...] = jnp.zeros_like(acc_ref)

    acc_ref[...] += jnp.dot(a_l_ref[...], a_r_ref[...],
                            preferred_element_type=jnp.float32)

    @pl.when(k == nk - 1)
    def _():
        a2_ref[...] = acc_ref[...]


def _a2_matmul(a):
    bm, bn, bk = 1024, 1024, 1024
    nk = N // bk
    grid = (N // bm, N // bn, nk)
    return pl.pallas_call(
        functools.partial(_a2_body, nk=nk),
        grid=grid,
        in_specs=[
            pl.BlockSpec((bm, bk), lambda i, j, k: (i, k)),
            pl.BlockSpec((bk, bn), lambda i, j, k: (k, j)),
        ],
        out_specs=pl.BlockSpec((bm, bn), lambda i, j, k: (i, j)),
        out_shape=jax.ShapeDtypeStruct((N, N), jnp.float32),
        scratch_shapes=[pltpu.VMEM((bm, bn), jnp.float32)],
    )(a, a)


def _sortable(d2):
    i = jax.lax.bitcast_convert_type(d2, jnp.int32)
    return i ^ ((i >> 31) & jnp.int32(0x7FFFFFFF))


def _keys_body(xi_ref, xj_ref, out_ref):
    xi = xi_ref[...]
    xj = xj_ref[...]
    sqi = jnp.sum(xi * xi, axis=1)
    sqj = jnp.sum(xj * xj, axis=1)
    g = jax.lax.dot_general(xi, xj, (((1,), (1,)), ((), ())),
                            preferred_element_type=jnp.float32)
    d2 = sqi[:, None] + sqj[None, :] - 2.0 * g
    out_ref[...] = _sortable(d2)


def _dist_keys(x):
    br = 256
    grid = (N // br, N // br)
    d = x.shape[1]
    return pl.pallas_call(
        _keys_body,
        grid=grid,
        in_specs=[
            pl.BlockSpec((br, d), lambda i, j: (i, 0)),
            pl.BlockSpec((br, d), lambda i, j: (j, 0)),
        ],
        out_specs=pl.BlockSpec((br, br), lambda i, j: (i, j)),
        out_shape=jax.ShapeDtypeStruct((N, N), jnp.int32),
    )(x, x)


def _thresh_body(dg_ref, a2_ref, keys_ref, tkey_ref, cls_ref, *, br):
    i = pl.program_id(0)
    a2 = a2_ref[...]
    rows = i * br + jax.lax.broadcasted_iota(jnp.int32, (br, N), 0)
    cols = jax.lax.broadcasted_iota(jnp.int32, (br, N), 1)
    nbr = (a2 > 0.0) & (rows != cols)
    dr = jnp.sum(nbr.astype(jnp.float32), axis=1).astype(jnp.int32)

    dgf_all = dg_ref[...]
    mean = jnp.mean(dgf_all)
    std = jnp.sqrt(jnp.mean((dgf_all - mean) ** 2))
    d_thres = mean + 2.0 * std

    dgf = dg_ref[pl.ds(i * br, br)]
    two_d = jnp.round(2.0 * dgf).astype(jnp.int32)
    cond1 = dgf > d_thres
    cond2 = (~cond1) & (two_d > dr)
    cond3 = (~cond1) & (~cond2)
    cls = jnp.where(cond1, 1, jnp.where(cond2, 2, 3)).astype(jnp.int32)

    kq = jnp.where(cond3 & (two_d < dr) & (two_d > 0), two_d, 0)
    k_idx = kq - 1

    keysm = jnp.where(nbr, keys_ref[...], INT32_MAX)

    def body(t, lo):
        b = 31 - t
        mid = lo + (jnp.int32(1) << b)
        cnt = jnp.sum((keysm < mid[:, None]).astype(jnp.int32), axis=1)
        return jnp.where(cnt <= k_idx, mid, lo)

    lo = jax.lax.fori_loop(0, 32, body, jnp.full((br,), INT32_MIN, jnp.int32))
    tkey_ref[...] = jnp.where(kq > 0, lo, INT32_MIN)
    cls_ref[...] = cls


def _thresholds(dgf, a2, keys):
    br = 256
    grid = (N // br,)
    return pl.pallas_call(
        functools.partial(_thresh_body, br=br),
        grid=grid,
        in_specs=[
            pl.BlockSpec((N,), lambda i: (0,)),
            pl.BlockSpec((br, N), lambda i: (i, 0)),
            pl.BlockSpec((br, N), lambda i: (i, 0)),
        ],
        out_specs=[
            pl.BlockSpec((br,), lambda i: (i,)),
            pl.BlockSpec((br,), lambda i: (i,)),
        ],
        out_shape=[
            jax.ShapeDtypeStruct((N,), jnp.int32),
            jax.ShapeDtypeStruct((N,), jnp.int32),
        ],
    )(dgf, a2, keys)


def _edge_combine(ridx, cidx, ew, arawv, a2v, keyv, tkey, cls, th1v, th2v):
    E = ew.shape[0]
    NW = 32
    CH = E // NW
    mesh = plsc.VectorSubcoreMesh(core_axis_name="c", subcore_axis_name="s")

    @functools.partial(
        pl.kernel,
        out_type=jax.ShapeDtypeStruct((E,), jnp.float32),
        mesh=mesh,
        scratch_types=[
            pltpu.VMEM((CH,), jnp.int32),
            pltpu.VMEM((CH,), jnp.int32),
            pltpu.VMEM((CH,), jnp.float32),
            pltpu.VMEM((CH,), jnp.int32),
            pltpu.VMEM((CH,), jnp.int32),
            pltpu.VMEM((CH,), jnp.int32),
            pltpu.VMEM((CH,), jnp.float32),
            pltpu.VMEM((CH,), jnp.float32),
            pltpu.VMEM((CH,), jnp.int32),
            pltpu.VMEM((CH,), jnp.float32),
            pltpu.VMEM((16,), jnp.float32),
            pltpu.VMEM((16,), jnp.float32),
            pltpu.SemaphoreType.DMA,
        ],
    )
    def body(r_hbm, c_hbm, w_hbm, araw_hbm, a2_hbm, key_hbm, tk_hbm, cls_hbm,
             th1_hbm, th2_hbm, out_hbm, rbuf, cbuf, wbuf, fbuf, tkbuf, clsbuf,
             gab, g2b, gkb, obuf, t1b, t2b, sem):
        wid = lax.axis_index("s") * 2 + lax.axis_index("c")
        base = wid * CH
        pltpu.sync_copy(r_hbm.at[pl.ds(base, CH)], rbuf)
        pltpu.sync_copy(c_hbm.at[pl.ds(base, CH)], cbuf)
        pltpu.sync_copy(w_hbm.at[pl.ds(base, CH)], wbuf)
        pltpu.sync_copy(th1_hbm, t1b)
        pltpu.sync_copy(th2_hbm, t2b)

        def idx_body(i, carry):
            rv = rbuf[pl.ds(i * 16, 16)]
            cv = cbuf[pl.ds(i * 16, 16)]
            fbuf[pl.ds(i * 16, 16)] = rv * 4096 + cv
            return carry

        lax.fori_loop(0, CH // 16, idx_body, 0)

        descs = []
        for j in range(CH // 128):
            sl = pl.ds(j * 128, 128)
            descs.append(pltpu.async_copy(
                tk_hbm.at[rbuf.at[sl]], tkbuf.at[sl], sem))
            descs.append(pltpu.async_copy(
                cls_hbm.at[rbuf.at[sl]], clsbuf.at[sl], sem))
            descs.append(pltpu.async_copy(
                araw_hbm.at[fbuf.at[sl]], gab.at[sl], sem))
            descs.append(pltpu.async_copy(
                a2_hbm.at[fbuf.at[sl]], g2b.at[sl], sem))
            descs.append(pltpu.async_copy(
                key_hbm.at[fbuf.at[sl]], gkb.at[sl], sem))
        for d in descs:
            d.wait()

        t1 = t1b[...]
        t2 = t2b[...]

        def ext_body(v, carry):
            p = v * 16
            rv = rbuf[pl.ds(p, 16)]
            cv = cbuf[pl.ds(p, 16)]
            arawe = gab[pl.ds(p, 16)]
            a2e = g2b[pl.ds(p, 16)]
            keye = gkb[pl.ds(p, 16)]
            tke = tkbuf[pl.ds(p, 16)]
            clse = clsbuf[pl.ds(p, 16)]
            wv = wbuf[pl.ds(p, 16)]
            ae = jnp.minimum(arawe, 1.0)
            bv = a2e - ae
            val1 = jnp.where(ae > 0.0, bv, 0.0)
            val3 = jnp.where((a2e > 0.0) & (keye > tke), 0.0, bv)
            val = jnp.where(clse == 1, val1,
                            jnp.where(clse == 2, bv, val3))
            val = jnp.where(rv == cv, 0.0, val)
            obuf[pl.ds(p, 16)] = jnp.maximum(t1 * wv + t2 * val, 0.0)
            return carry

        lax.fori_loop(0, CH // 16, ext_body, 0)

        pltpu.sync_copy(obuf, out_hbm.at[pl.ds(base, CH)])

    return body(ridx, cidx, ew, arawv, a2v, keyv, tkey, cls, th1v, th2v)


def kernel(edge_index, edge_weight, features, theta_1, theta_2):
    row, col = edge_index[0], edge_index[1]
    ew_const = jax.lax.stop_gradient(edge_weight)
    x = jax.lax.stop_gradient(features)

    araw = jnp.zeros((N, N), jnp.float32).at[row, col].add(ew_const)
    a, dgf = _clip_a(araw)
    a2 = _a2_matmul(a)
    keys = _dist_keys(x)
    tkey, cls = _thresholds(dgf, a2, keys)

    th1v = jnp.broadcast_to(theta_1, (16,))
    th2v = jnp.broadcast_to(theta_2, (16,))
    return _edge_combine(row, col, edge_weight,
                         araw.reshape(N * N),
                         a2.reshape(N * N),
                         keys.reshape(N * N),
                         tkey, cls, th1v, th2v)
